# Initial kernel scaffold; baseline (speedup 1.0000x reference)
import jax
import jax.numpy as jnp
from jax.experimental import pallas as pl


def kernel(x, table, lora_A, lora_B):
    raise NotImplementedError("write your pallas kernel here")


# speedup vs baseline: 4.2775x; 4.2775x over previous
import functools

import jax
import jax.numpy as jnp
from jax import lax
from jax.experimental import pallas as pl
from jax.experimental.pallas import tpu as pltpu
from jax.experimental.pallas import tpu_sc as plsc

_EMBED_DIM = 64
_RANK = 16
_SCALING = 32.0 / 16.0

_NC = 2
_NS = 16
_NW = _NC * _NS
_CH = 128


def _sc_lookup(x_flat, table, lora_a, lora_bs):
    n = x_flat.shape[0]
    rows_per_w = n // _NW
    n_chunks = rows_per_w // _CH
    d = _EMBED_DIM
    r_rank = _RANK

    mesh = plsc.VectorSubcoreMesh(core_axis_name="c", subcore_axis_name="s")

    @functools.partial(
        pl.kernel,
        mesh=mesh,
        out_type=jax.ShapeDtypeStruct((n, d), jnp.float32),
        scratch_types=[
            pltpu.VMEM((_CH,), jnp.int32),
            pltpu.VMEM((_CH, d), jnp.float32),
            pltpu.VMEM((_CH, r_rank), jnp.float32),
            pltpu.VMEM((_CH, d), jnp.float32),
            pltpu.VMEM((r_rank, d), jnp.float32),
            pltpu.SemaphoreType.DMA,
            pltpu.SemaphoreType.DMA,
        ],
        compiler_params=pltpu.CompilerParams(use_tc_tiling_on_sc=False),
    )
    def k(x_hbm, tab_hbm, la_hbm, b_hbm, out_hbm,
          idx_v, base_v, lora_v, out_v, b_v, sem0, sem1):
        wid = lax.axis_index("s") * _NC + lax.axis_index("c")
        base_off = wid * rows_per_w
        pltpu.sync_copy(b_hbm, b_v)

        def chunk_body(kk, carry):
            off = base_off + kk * _CH
            pltpu.sync_copy(x_hbm.at[pl.ds(off, _CH)], idx_v)
            cp0 = pltpu.async_copy(tab_hbm.at[idx_v], base_v, sem0)
            cp1 = pltpu.async_copy(la_hbm.at[idx_v], lora_v, sem1)
            cp0.wait()
            cp1.wait()

            def row_body(i, c2):
                lv = lora_v[i, :]
                for c in range(d // 16):
                    acc = base_v[i, pl.ds(c * 16, 16)]
                    for r in range(r_rank):
                        acc = acc + lv[r] * b_v[r, pl.ds(c * 16, 16)]
                    out_v[i, pl.ds(c * 16, 16)] = acc
                return c2

            lax.fori_loop(0, _CH, row_body, 0)
            pltpu.sync_copy(out_v, out_hbm.at[pl.ds(off, _CH)])
            return carry

        lax.fori_loop(0, n_chunks, chunk_body, 0)

    return k(x_flat, table, lora_a, lora_bs)


def kernel(x, table, lora_A, lora_B):
    b, l = x.shape
    x_flat = x.reshape(-1).astype(jnp.int32)
    bs = (lora_B * _SCALING).astype(jnp.float32)
    out = _sc_lookup(x_flat, table, lora_A, bs)
    return out.reshape(b, l, _EMBED_DIM)


# speedup vs baseline: 6.5794x; 1.5381x over previous
import functools

import jax
import jax.numpy as jnp
from jax import lax
from jax.experimental import pallas as pl
from jax.experimental.pallas import tpu as pltpu
from jax.experimental.pallas import tpu_sc as plsc

_EMBED_DIM = 64
_RANK = 16
_SCALING = 32.0 / 16.0

_NC = 2
_NS = 16
_NW = _NC * _NS
_CH = 128
_NBUF = 4


def _sc_lookup(x_flat, table, lora_a, lora_bs):
    n = x_flat.shape[0]
    rows_per_w = n // _NW
    n_chunks = rows_per_w // _CH
    n_groups = n_chunks // _NBUF
    d = _EMBED_DIM
    r_rank = _RANK

    mesh = plsc.VectorSubcoreMesh(core_axis_name="c", subcore_axis_name="s")

    @functools.partial(
        pl.kernel,
        mesh=mesh,
        out_type=jax.ShapeDtypeStruct((n, d), jnp.float32),
        scratch_types=[
            pltpu.VMEM((rows_per_w,), jnp.int32),
            pltpu.VMEM((_NBUF, _CH, d), jnp.float32),
            pltpu.VMEM((_NBUF, _CH, r_rank), jnp.float32),
            pltpu.VMEM((_NBUF, _CH, d), jnp.float32),
            pltpu.VMEM((r_rank, d), jnp.float32),
            [pltpu.SemaphoreType.DMA] * _NBUF,
            [pltpu.SemaphoreType.DMA] * _NBUF,
            [pltpu.SemaphoreType.DMA] * _NBUF,
        ],
        compiler_params=pltpu.CompilerParams(use_tc_tiling_on_sc=False),
    )
    def k(x_hbm, tab_hbm, la_hbm, b_hbm, out_hbm,
          idx_v, base_v, lora_v, out_v, b_v, sem_t, sem_l, sem_o):
        wid = lax.axis_index("s") * _NC + lax.axis_index("c")
        base_off = wid * rows_per_w
        pltpu.sync_copy(b_hbm, b_v)
        pltpu.sync_copy(x_hbm.at[pl.ds(base_off, rows_per_w)], idx_v)

        def compute(slot):
            bs = base_v.at[slot]
            ls = lora_v.at[slot]
            os_ = out_v.at[slot]
            for half in range(2):
                bh = [b_v[r, pl.ds(half * 32 + cc * 16, 16)]
                      for r in range(r_rank) for cc in range(2)]

                def row_body(i, carry):
                    lv = ls[i, :]
                    a0 = bs[i, pl.ds(half * 32, 16)]
                    a1 = bs[i, pl.ds(half * 32 + 16, 16)]
                    s8 = lv[8]
                    b0 = s8 * carry[16]
                    b1 = s8 * carry[17]
                    for r in range(8):
                        s = lv[r]
                        a0 = a0 + s * carry[2 * r]
                        a1 = a1 + s * carry[2 * r + 1]
                    for r in range(9, r_rank):
                        s = lv[r]
                        b0 = b0 + s * carry[2 * r]
                        b1 = b1 + s * carry[2 * r + 1]
                    os_[i, pl.ds(half * 32, 16)] = a0 + b0
                    os_[i, pl.ds(half * 32 + 16, 16)] = a1 + b1
                    return carry

                lax.fori_loop(0, _CH, row_body, tuple(bh))

        def group_body(q, carry):
            k0 = q * _NBUF
            gathers = []
            for s in range(_NBUF):
                iref = idx_v.at[pl.ds((k0 + s) * _CH, _CH)]
                cp_t = pltpu.async_copy(tab_hbm.at[iref], base_v.at[s],
                                        sem_t[s])
                cp_l = pltpu.async_copy(la_hbm.at[iref], lora_v.at[s],
                                        sem_l[s])
                gathers.append((cp_t, cp_l))
            stores = []
            for s in range(_NBUF):
                cp_t, cp_l = gathers[s]
                cp_t.wait()
                cp_l.wait()
                compute(s)
                stores.append(pltpu.async_copy(
                    out_v.at[s],
                    out_hbm.at[pl.ds(base_off + (k0 + s) * _CH, _CH)],
                    sem_o[s]))
            for cp in stores:
                cp.wait()
            return carry

        lax.fori_loop(0, n_groups, group_body, 0)

    return k(x_flat, table, lora_a, lora_bs)


def kernel(x, table, lora_A, lora_B):
    b, l = x.shape
    x_flat = x.reshape(-1).astype(jnp.int32)
    bs = (lora_B * _SCALING).astype(jnp.float32)
    out = _sc_lookup(x_flat, table, lora_A, bs)
    return out.reshape(b, l, _EMBED_DIM)


# speedup vs baseline: 9.3689x; 1.4240x over previous
import functools

import jax
import jax.numpy as jnp
from jax import lax
from jax.experimental import pallas as pl
from jax.experimental.pallas import tpu as pltpu
from jax.experimental.pallas import tpu_sc as plsc

_EMBED_DIM = 64
_RANK = 16
_SCALING = 32.0 / 16.0

_NC = 2
_NS = 16
_NW = _NC * _NS
_CH = 128
_NBUF = 4
_FBLK = 4000


def _fuse_body(tab_ref, la_ref, b_ref, out_ref):
    f = tab_ref[...] + jnp.dot(la_ref[...], b_ref[...],
                               preferred_element_type=jnp.float32)
    out_ref[...] = jnp.concatenate([f, f], axis=1)


def _fused_dup_table(table, lora_a, lora_bs):
    nv = table.shape[0]
    grid = nv // _FBLK
    return pl.pallas_call(
        _fuse_body,
        grid=(grid,),
        in_specs=[
            pl.BlockSpec((_FBLK, _EMBED_DIM), lambda i: (i, 0)),
            pl.BlockSpec((_FBLK, _RANK), lambda i: (i, 0)),
            pl.BlockSpec((_RANK, _EMBED_DIM), lambda i: (0, 0)),
        ],
        out_specs=pl.BlockSpec((_FBLK, 2 * _EMBED_DIM), lambda i: (i, 0)),
        out_shape=jax.ShapeDtypeStruct((nv, 2 * _EMBED_DIM), jnp.float32),
    )(table, lora_a, lora_bs)


def _sc_lookup(x_flat, dup_tab):
    n = x_flat.shape[0]
    rows_per_w = n // _NW
    n_chunks = rows_per_w // _CH
    n_groups = n_chunks // _NBUF
    d2 = 2 * _EMBED_DIM
    half_ch = _CH // 2

    mesh = plsc.VectorSubcoreMesh(core_axis_name="c", subcore_axis_name="s")

    @functools.partial(
        pl.kernel,
        mesh=mesh,
        out_type=jax.ShapeDtypeStruct((n // 2, d2), jnp.float32),
        scratch_types=[
            pltpu.VMEM((rows_per_w,), jnp.int32),
            pltpu.VMEM((_NBUF, _CH, d2), jnp.float32),
            [pltpu.SemaphoreType.DMA] * _NBUF,
            [pltpu.SemaphoreType.DMA] * _NBUF,
        ],
    )
    def k(x_hbm, tab_hbm, out_hbm, idx_v, gath_v, sem_g, sem_o):
        wid = lax.axis_index("s") * _NC + lax.axis_index("c")
        base_off = wid * rows_per_w
        out_base = wid * (rows_per_w // 2)
        pltpu.sync_copy(x_hbm.at[pl.ds(base_off, rows_per_w)], idx_v)

        def repack(slot):
            g = gath_v.at[slot]

            def row_body(j, carry):
                lo = [g[2 * j, pl.ds(c * 16, 16)] for c in range(4)]
                hi = [g[2 * j + 1, pl.ds(64 + c * 16, 16)] for c in range(4)]
                for c in range(4):
                    g[j, pl.ds(c * 16, 16)] = lo[c]
                for c in range(4):
                    g[j, pl.ds(64 + c * 16, 16)] = hi[c]
                return carry

            lax.fori_loop(0, half_ch, row_body, 0)

        def group_body(q, carry):
            k0 = q * _NBUF
            gathers = []
            for s in range(_NBUF):
                iref = idx_v.at[pl.ds((k0 + s) * _CH, _CH)]
                gathers.append(pltpu.async_copy(
                    tab_hbm.at[iref], gath_v.at[s], sem_g[s]))
            stores = []
            for s in range(_NBUF):
                gathers[s].wait()
                repack(s)
                stores.append(pltpu.async_copy(
                    gath_v.at[s, pl.ds(0, half_ch)],
                    out_hbm.at[pl.ds(out_base + (k0 + s) * half_ch, half_ch)],
                    sem_o[s]))
            for cp in stores:
                cp.wait()
            return carry

        lax.fori_loop(0, n_groups, group_body, 0)

    return k(x_flat, dup_tab)


def kernel(x, table, lora_A, lora_B):
    b, l = x.shape
    x_flat = x.reshape(-1).astype(jnp.int32)
    bs = (lora_B * _SCALING).astype(jnp.float32)
    dup_tab = _fused_dup_table(table, lora_A, bs)
    out = _sc_lookup(x_flat, dup_tab)
    return out.reshape(b, l, _EMBED_DIM)


# speedup vs baseline: 11.2696x; 1.2029x over previous
import functools

import jax
import jax.numpy as jnp
from jax import lax
from jax.experimental import pallas as pl
from jax.experimental.pallas import tpu as pltpu
from jax.experimental.pallas import tpu_sc as plsc

_EMBED_DIM = 64
_RANK = 16
_SCALING = 32.0 / 16.0

_NC = 2
_NS = 16
_NW = _NC * _NS
_CH = 128
_NBUF = 4
_FBLK = 4000


_V1 = 1000
_V2 = 1000
_FJ = 8


def _fuse_body(tab_t_ref, la_t_ref, b_ref, eye_ref, out_ref):
    for j in range(_FJ):
        delta_t = lax.dot_general(la_t_ref[:, j, :], b_ref[...],
                                  (((0,), (0,)), ((), ())),
                                  preferred_element_type=jnp.float32)
        tab_t = lax.dot_general(tab_t_ref[:, j, :], eye_ref[...],
                                (((0,), (0,)), ((), ())),
                                preferred_element_type=jnp.float32)
        ft = tab_t + delta_t
        out_ref[j] = jnp.concatenate([ft, ft], axis=1)


def _fused_dup_table(table_t3, lora_a_t3, lora_bs, eye):
    grid = _V1 // _FJ
    return pl.pallas_call(
        _fuse_body,
        grid=(grid,),
        in_specs=[
            pl.BlockSpec((_EMBED_DIM, _FJ, _V2), lambda i: (0, i, 0)),
            pl.BlockSpec((_RANK, _FJ, _V2), lambda i: (0, i, 0)),
            pl.BlockSpec((_RANK, _EMBED_DIM), lambda i: (0, 0)),
            pl.BlockSpec((_EMBED_DIM, _EMBED_DIM), lambda i: (0, 0)),
        ],
        out_specs=pl.BlockSpec((_FJ, _V2, 2 * _EMBED_DIM), lambda i: (i, 0, 0)),
        out_shape=jax.ShapeDtypeStruct((_V1, _V2, 2 * _EMBED_DIM), jnp.float32),
    )(table_t3, lora_a_t3, lora_bs, eye)


def _sc_lookup(x_flat, dup_tab):
    n = x_flat.shape[0]
    rows_per_w = n // _NW
    n_chunks = rows_per_w // _CH
    n_groups = n_chunks // _NBUF
    d2 = 2 * _EMBED_DIM

    mesh = plsc.VectorSubcoreMesh(core_axis_name="c", subcore_axis_name="s")

    @functools.partial(
        pl.kernel,
        mesh=mesh,
        out_type=jax.ShapeDtypeStruct((n, d2), jnp.float32),
        scratch_types=[
            pltpu.VMEM((rows_per_w,), jnp.int32),
            pltpu.VMEM((_NBUF, _CH, d2), jnp.float32),
            [pltpu.SemaphoreType.DMA] * _NBUF,
            [pltpu.SemaphoreType.DMA] * _NBUF,
        ],
    )
    def k(x_hbm, tab_hbm, out_hbm, idx_v, gath_v, sem_g, sem_o):
        wid = lax.axis_index("s") * _NC + lax.axis_index("c")
        base_off = wid * rows_per_w
        pltpu.sync_copy(x_hbm.at[pl.ds(base_off, rows_per_w)], idx_v)

        def group_body(q, carry):
            k0 = q * _NBUF
            gathers = []
            for s in range(_NBUF):
                iref = idx_v.at[pl.ds((k0 + s) * _CH, _CH)]
                gathers.append(pltpu.async_copy(
                    tab_hbm.at[iref], gath_v.at[s], sem_g[s]))
            stores = []
            for s in range(_NBUF):
                gathers[s].wait()
                stores.append(pltpu.async_copy(
                    gath_v.at[s],
                    out_hbm.at[pl.ds(base_off + (k0 + s) * _CH, _CH)],
                    sem_o[s]))
            for cp in stores:
                cp.wait()
            return carry

        lax.fori_loop(0, n_groups, group_body, 0)

    return k(x_flat, dup_tab)


def kernel(x, table, lora_A, lora_B):
    b, l = x.shape
    x_flat = x.reshape(-1).astype(jnp.int32)
    bs = (lora_B * _SCALING).astype(jnp.float32)
    eye = jnp.eye(_EMBED_DIM, dtype=jnp.float32)
    dup3 = _fused_dup_table(table.T.reshape(_EMBED_DIM, _V1, _V2),
                            lora_A.T.reshape(_RANK, _V1, _V2), bs, eye)
    dup_tab = dup3.reshape(_V1 * _V2, 2 * _EMBED_DIM)
    out = _sc_lookup(x_flat, dup_tab)
    return out.reshape(b, l, 2 * _EMBED_DIM)[:, :, :_EMBED_DIM]


# speedup vs baseline: 19.2265x; 1.7060x over previous
import functools

import jax
import jax.numpy as jnp
from jax import lax
from jax.experimental import pallas as pl
from jax.experimental.pallas import tpu as pltpu
from jax.experimental.pallas import tpu_sc as plsc

_EMBED_DIM = 64
_RANK = 16
_SCALING = 32.0 / 16.0

_NC = 2
_NS = 16
_NW = _NC * _NS
_CH = 128
_NBUF = 4
_FBLK = 4000


_VB = 8192


def _fuse_body(tab_t_ref, la_t_ref, b_ref, eye_ref, out_ref):
    delta_t = lax.dot_general(la_t_ref[...], b_ref[...],
                              (((0,), (0,)), ((), ())),
                              preferred_element_type=jnp.float32)
    tab_t = lax.dot_general(tab_t_ref[...], eye_ref[...],
                            (((0,), (0,)), ((), ())),
                            preferred_element_type=jnp.float32)
    ft = tab_t + delta_t
    out_ref[...] = jnp.concatenate([ft, ft], axis=1)


def _fused_dup_table(table_t, lora_a_t, lora_bs, eye):
    nv = table_t.shape[1]
    grid = pl.cdiv(nv, _VB)
    return pl.pallas_call(
        _fuse_body,
        grid=(grid,),
        in_specs=[
            pl.BlockSpec((_EMBED_DIM, _VB), lambda i: (0, i)),
            pl.BlockSpec((_RANK, _VB), lambda i: (0, i)),
            pl.BlockSpec((_RANK, _EMBED_DIM), lambda i: (0, 0)),
            pl.BlockSpec((_EMBED_DIM, _EMBED_DIM), lambda i: (0, 0)),
        ],
        out_specs=pl.BlockSpec((_VB, 2 * _EMBED_DIM), lambda i: (i, 0)),
        out_shape=jax.ShapeDtypeStruct((nv, 2 * _EMBED_DIM), jnp.float32),
    )(table_t, lora_a_t, lora_bs, eye)


def _sc_lookup(x_flat, dup_tab):
    n = x_flat.shape[0]
    rows_per_w = n // _NW
    n_chunks = rows_per_w // _CH
    n_groups = n_chunks // _NBUF
    d2 = 2 * _EMBED_DIM

    mesh = plsc.VectorSubcoreMesh(core_axis_name="c", subcore_axis_name="s")

    @functools.partial(
        pl.kernel,
        mesh=mesh,
        out_type=jax.ShapeDtypeStruct((n, d2), jnp.float32),
        scratch_types=[
            pltpu.VMEM((rows_per_w,), jnp.int32),
            pltpu.VMEM((_NBUF, _CH, d2), jnp.float32),
            [pltpu.SemaphoreType.DMA] * _NBUF,
            [pltpu.SemaphoreType.DMA] * _NBUF,
        ],
    )
    def k(x_hbm, tab_hbm, out_hbm, idx_v, gath_v, sem_g, sem_o):
        wid = lax.axis_index("s") * _NC + lax.axis_index("c")
        base_off = wid * rows_per_w
        pltpu.sync_copy(x_hbm.at[pl.ds(base_off, rows_per_w)], idx_v)

        def group_body(q, carry):
            k0 = q * _NBUF
            gathers = []
            for s in range(_NBUF):
                iref = idx_v.at[pl.ds((k0 + s) * _CH, _CH)]
                gathers.append(pltpu.async_copy(
                    tab_hbm.at[iref], gath_v.at[s], sem_g[s]))
            stores = []
            for s in range(_NBUF):
                gathers[s].wait()
                stores.append(pltpu.async_copy(
                    gath_v.at[s],
                    out_hbm.at[pl.ds(base_off + (k0 + s) * _CH, _CH)],
                    sem_o[s]))
            for cp in stores:
                cp.wait()
            return carry

        lax.fori_loop(0, n_groups, group_body, 0)

    return k(x_flat, dup_tab)


def kernel(x, table, lora_A, lora_B):
    b, l = x.shape
    x_flat = x.reshape(-1).astype(jnp.int32)
    bs = (lora_B * _SCALING).astype(jnp.float32)
    eye = jnp.eye(_EMBED_DIM, dtype=jnp.float32)
    dup_tab = _fused_dup_table(table.T, lora_A.T, bs, eye)
    out = _sc_lookup(x_flat, dup_tab)
    return out.reshape(b, l, 2 * _EMBED_DIM)[:, :, :_EMBED_DIM]


# speedup vs baseline: 19.2734x; 1.0024x over previous
import functools

import jax
import jax.numpy as jnp
from jax import lax
from jax.experimental import pallas as pl
from jax.experimental.pallas import tpu as pltpu
from jax.experimental.pallas import tpu_sc as plsc

_EMBED_DIM = 64
_RANK = 16
_SCALING = 32.0 / 16.0

_NC = 2
_NS = 16
_NW = _NC * _NS
_CH = 128
_NBUF = 4
_FBLK = 4000


_VB = 16384


def _fuse_body(tab_t_ref, la_t_ref, b_ref, eye_ref, out_ref):
    delta_t = lax.dot_general(la_t_ref[...], b_ref[...],
                              (((0,), (0,)), ((), ())),
                              preferred_element_type=jnp.float32)
    tab_t = lax.dot_general(tab_t_ref[...], eye_ref[...],
                            (((0,), (0,)), ((), ())),
                            preferred_element_type=jnp.float32)
    ft = tab_t + delta_t
    out_ref[...] = jnp.concatenate([ft, ft], axis=1)


def _fused_dup_table(table_t, lora_a_t, lora_bs, eye):
    nv = table_t.shape[1]
    grid = pl.cdiv(nv, _VB)
    return pl.pallas_call(
        _fuse_body,
        grid=(grid,),
        in_specs=[
            pl.BlockSpec((_EMBED_DIM, _VB), lambda i: (0, i)),
            pl.BlockSpec((_RANK, _VB), lambda i: (0, i)),
            pl.BlockSpec((_RANK, _EMBED_DIM), lambda i: (0, 0)),
            pl.BlockSpec((_EMBED_DIM, _EMBED_DIM), lambda i: (0, 0)),
        ],
        out_specs=pl.BlockSpec((_VB, 2 * _EMBED_DIM), lambda i: (i, 0)),
        out_shape=jax.ShapeDtypeStruct((nv, 2 * _EMBED_DIM), jnp.float32),
    )(table_t, lora_a_t, lora_bs, eye)


def _sc_lookup(x_flat, dup_tab):
    n = x_flat.shape[0]
    rows_per_w = n // _NW
    n_chunks = rows_per_w // _CH
    n_groups = n_chunks // _NBUF
    d2 = 2 * _EMBED_DIM

    mesh = plsc.VectorSubcoreMesh(core_axis_name="c", subcore_axis_name="s")

    @functools.partial(
        pl.kernel,
        mesh=mesh,
        out_type=jax.ShapeDtypeStruct((n, d2), jnp.float32),
        scratch_types=[
            pltpu.VMEM((rows_per_w,), jnp.int32),
            pltpu.VMEM((_NBUF, _CH, d2), jnp.float32),
            [pltpu.SemaphoreType.DMA] * _NBUF,
            [pltpu.SemaphoreType.DMA] * _NBUF,
        ],
    )
    def k(x_hbm, tab_hbm, out_hbm, idx_v, gath_v, sem_g, sem_o):
        wid = lax.axis_index("s") * _NC + lax.axis_index("c")
        base_off = wid * rows_per_w
        pltpu.sync_copy(x_hbm.at[pl.ds(base_off, rows_per_w)], idx_v)

        def group_body(q, carry):
            k0 = q * _NBUF
            gathers = []
            for s in range(_NBUF):
                iref = idx_v.at[pl.ds((k0 + s) * _CH, _CH)]
                gathers.append(pltpu.async_copy(
                    tab_hbm.at[iref], gath_v.at[s], sem_g[s]))
            stores = []
            for s in range(_NBUF):
                gathers[s].wait()
                stores.append(pltpu.async_copy(
                    gath_v.at[s],
                    out_hbm.at[pl.ds(base_off + (k0 + s) * _CH, _CH)],
                    sem_o[s]))
            for cp in stores:
                cp.wait()
            return carry

        lax.fori_loop(0, n_groups, group_body, 0)

    return k(x_flat, dup_tab)


def kernel(x, table, lora_A, lora_B):
    b, l = x.shape
    x_flat = x.reshape(-1).astype(jnp.int32)
    bs = (lora_B * _SCALING).astype(jnp.float32)
    eye = jnp.eye(_EMBED_DIM, dtype=jnp.float32)
    dup_tab = _fused_dup_table(table.T, lora_A.T, bs, eye)
    out = _sc_lookup(x_flat, dup_tab)
    return out.reshape(b, l, 2 * _EMBED_DIM)[:, :, :_EMBED_DIM]
